# Initial kernel scaffold; baseline (speedup 1.0000x reference)
import jax
import jax.numpy as jnp
from jax.experimental import pallas as pl


def kernel(hidden_states, weight):
    raise NotImplementedError("write your pallas kernel here")


# speedup vs baseline: 1.1028x; 1.1028x over previous
import jax
import jax.numpy as jnp
from jax.experimental import pallas as pl
from jax.experimental.pallas import tpu as pltpu

_HIDDEN = 4096
_N_EXPERTS = 64
_TOP_K = 8
_BLOCK = 512


def _router_body(x_ref, w_ref, logits_ref, topv_ref, topi_ref):
    x = x_ref[...]
    w = w_ref[...]
    logits = jax.lax.dot_general(
        x, w, (((1,), (1,)), ((), ())), preferred_element_type=jnp.float32
    )
    logits_ref[...] = logits

    m = jnp.max(logits, axis=-1, keepdims=True)
    e = jnp.exp(logits - m)
    probs = e / jnp.sum(e, axis=-1, keepdims=True)

    col = jax.lax.broadcasted_iota(jnp.int32, probs.shape, 1)
    p = probs
    vals = []
    idxs = []
    for _ in range(_TOP_K):
        mk = jnp.max(p, axis=-1, keepdims=True)
        is_max = p == mk
        ik = jnp.min(
            jnp.where(is_max, col, _N_EXPERTS), axis=-1, keepdims=True
        )
        vals.append(mk)
        idxs.append(ik)
        p = jnp.where(col == ik, -1.0, p)

    top_v = jnp.concatenate(vals, axis=-1)
    top_i = jnp.concatenate(idxs, axis=-1)
    topv_ref[...] = top_v / jnp.sum(top_v, axis=-1, keepdims=True)
    topi_ref[...] = top_i


def kernel(hidden_states, weight):
    x = hidden_states.reshape(-1, _HIDDEN)
    n_tokens = x.shape[0]
    grid = (n_tokens // _BLOCK,)

    out_shapes = (
        jax.ShapeDtypeStruct((n_tokens, _N_EXPERTS), jnp.float32),
        jax.ShapeDtypeStruct((n_tokens, _TOP_K), jnp.float32),
        jax.ShapeDtypeStruct((n_tokens, _TOP_K), jnp.int32),
    )
    return pl.pallas_call(
        _router_body,
        grid=grid,
        in_specs=[
            pl.BlockSpec((_BLOCK, _HIDDEN), lambda i: (i, 0)),
            pl.BlockSpec((_N_EXPERTS, _HIDDEN), lambda i: (0, 0)),
        ],
        out_specs=(
            pl.BlockSpec((_BLOCK, _N_EXPERTS), lambda i: (i, 0)),
            pl.BlockSpec((_BLOCK, _TOP_K), lambda i: (i, 0)),
            pl.BlockSpec((_BLOCK, _TOP_K), lambda i: (i, 0)),
        ),
        out_shape=out_shapes,
        compiler_params=pltpu.CompilerParams(
            dimension_semantics=("arbitrary",),
        ),
    )(x, weight)


# speedup vs baseline: 1.2484x; 1.1320x over previous
import jax
import jax.numpy as jnp
from jax.experimental import pallas as pl
from jax.experimental.pallas import tpu as pltpu

_HIDDEN = 4096
_N_EXPERTS = 64
_TOP_K = 8
_BLOCK = 1024


def _router_body(x_ref, w_ref, logits_ref, topv_ref, topi_ref):
    x = x_ref[...]
    w = w_ref[...]
    logits = jax.lax.dot_general(
        x, w, (((1,), (1,)), ((), ())), preferred_element_type=jnp.float32
    )
    logits_ref[...] = logits

    m = jnp.max(logits, axis=-1, keepdims=True)
    e = jnp.exp(logits - m)
    probs = e / jnp.sum(e, axis=-1, keepdims=True)

    col = jax.lax.broadcasted_iota(jnp.int32, probs.shape, 1)
    p = probs
    vals = []
    idxs = []
    for _ in range(_TOP_K):
        mk = jnp.max(p, axis=-1, keepdims=True)
        is_max = p == mk
        ik = jnp.min(
            jnp.where(is_max, col, _N_EXPERTS), axis=-1, keepdims=True
        )
        vals.append(mk)
        idxs.append(ik)
        p = jnp.where(col == ik, -1.0, p)

    top_v = jnp.concatenate(vals, axis=-1)
    top_i = jnp.concatenate(idxs, axis=-1)
    topv_ref[...] = top_v / jnp.sum(top_v, axis=-1, keepdims=True)
    topi_ref[...] = top_i


def kernel(hidden_states, weight):
    x = hidden_states.reshape(-1, _HIDDEN)
    n_tokens = x.shape[0]
    grid = (n_tokens // _BLOCK,)

    out_shapes = (
        jax.ShapeDtypeStruct((n_tokens, _N_EXPERTS), jnp.float32),
        jax.ShapeDtypeStruct((n_tokens, _TOP_K), jnp.float32),
        jax.ShapeDtypeStruct((n_tokens, _TOP_K), jnp.int32),
    )
    return pl.pallas_call(
        _router_body,
        grid=grid,
        in_specs=[
            pl.BlockSpec((_BLOCK, _HIDDEN), lambda i: (i, 0)),
            pl.BlockSpec((_N_EXPERTS, _HIDDEN), lambda i: (0, 0)),
        ],
        out_specs=(
            pl.BlockSpec((_BLOCK, _N_EXPERTS), lambda i: (i, 0)),
            pl.BlockSpec((_BLOCK, _TOP_K), lambda i: (i, 0)),
            pl.BlockSpec((_BLOCK, _TOP_K), lambda i: (i, 0)),
        ),
        out_shape=out_shapes,
        compiler_params=pltpu.CompilerParams(
            dimension_semantics=("arbitrary",),
        ),
    )(x, weight)
